# baseline (speedup 1.0000x reference)
import functools

import jax
import jax.numpy as jnp
from jax import lax
from jax.experimental import pallas as pl
from jax.experimental.pallas import tpu as pltpu
from jax.experimental.pallas import tpu_sc as plsc

B, S, H = 32, 8, 1024
V = 100000
ROWS = B * S
NC, NS = 2, 16
NW = NC * NS
RPW = ROWS // NW
ROW_BLK = 64
MAGIC = 12582912.0

_mesh = plsc.VectorSubcoreMesh(core_axis_name="c", subcore_axis_name="s")


def _zero_body(out_ref):
    out_ref[...] = jnp.zeros((ROW_BLK, V), jnp.float32)


_tc_zeros = pl.pallas_call(
    _zero_body,
    out_shape=jax.ShapeDtypeStruct((ROWS, V), jnp.float32),
    grid=(ROWS // ROW_BLK,),
    out_specs=pl.BlockSpec((ROW_BLK, V), lambda i: (i, 0)),
)


@functools.partial(
    pl.kernel,
    mesh=_mesh,
    out_type=(),
    scratch_types=[
        pltpu.VMEM((RPW, 128), jnp.float32),
        pltpu.VMEM((RPW, RPW, 128), jnp.float32),
        pltpu.SemaphoreType.DMA,
    ],
)
def _sc_scatter(hs_hbm, out_hbm, hsv, wbuf, wsem):
    wid = lax.axis_index("s") * NC + lax.axis_index("c")
    base = wid * RPW

    pltpu.sync_copy(hs_hbm.at[pl.ds(base, RPW), pl.ds(0, 128)], hsv)
    lane = lax.iota(jnp.int32, 16)
    toks = []
    colbs = []
    for r in range(RPW):
        x = hsv[r, pl.ds(0, 16)][0]
        rr = (x + MAGIC) - MAGIC
        rr = jnp.minimum(jnp.maximum(rr, 0.0), float(V - 1))
        tok = rr.astype(jnp.int32)
        toks.append(tok)
        colb = tok - jnp.bitwise_and(tok, 127)
        colbs.append(pl.multiple_of(colb, 128))

    copies = []
    for k in range(RPW):
        for i in range(RPW):
            for j in range(8):
                col = colbs[k] + (lane + j * 16)
                wbuf[k, i, pl.ds(j * 16, 16)] = jnp.where(
                    col == toks[i], jnp.float32(5.0), jnp.float32(0.0)
                )
        dst = out_hbm.at[pl.ds(base, RPW), pl.ds(colbs[k], 128)]
        copies.append(pltpu.async_copy(wbuf.at[k], dst, wsem))
    for c in copies:
        c.wait()


def kernel(hidden_states):
    hs = hidden_states.reshape(ROWS, H)
    logits = _tc_zeros()
    buf = jax.new_ref(logits)
    _sc_scatter(hs, buf)
    return buf[...].reshape(B, S, V)


# speedup vs baseline: 1.0466x; 1.0466x over previous
import functools

import jax
import jax.numpy as jnp
from jax import lax
from jax.experimental import pallas as pl
from jax.experimental.pallas import tpu as pltpu
from jax.experimental.pallas import tpu_sc as plsc

B, S, H = 32, 8, 1024
V = 100000
ROWS = B * S
NC, NS = 2, 16
NW = NC * NS
RPW = ROWS // NW
ROW_BLK = 16
MAGIC = 12582912.0

_mesh = plsc.VectorSubcoreMesh(core_axis_name="c", subcore_axis_name="s")


def _zero_body(out_ref):
    out_ref[...] = jnp.zeros((ROW_BLK, V), jnp.float32)


_tc_zeros = pl.pallas_call(
    _zero_body,
    out_shape=jax.ShapeDtypeStruct((ROWS, V), jnp.float32),
    grid=(ROWS // ROW_BLK,),
    out_specs=pl.BlockSpec((ROW_BLK, V), lambda i: (i, 0)),
)


@functools.partial(
    pl.kernel,
    mesh=_mesh,
    out_type=(),
    scratch_types=[
        pltpu.VMEM((RPW, 128), jnp.float32),
        pltpu.VMEM((RPW, RPW, 128), jnp.float32),
        pltpu.SemaphoreType.DMA,
    ],
)
def _sc_scatter(hs_hbm, out_hbm, hsv, wbuf, wsem):
    wid = lax.axis_index("s") * NC + lax.axis_index("c")
    base = wid * RPW

    pltpu.sync_copy(hs_hbm.at[pl.ds(base, RPW), pl.ds(0, 128)], hsv)
    lane = lax.iota(jnp.int32, 16)
    toks = []
    colbs = []
    for r in range(RPW):
        x = hsv[r, pl.ds(0, 16)][0]
        rr = (x + MAGIC) - MAGIC
        rr = jnp.minimum(jnp.maximum(rr, 0.0), float(V - 1))
        tok = rr.astype(jnp.int32)
        toks.append(tok)
        colb = tok - jnp.bitwise_and(tok, 127)
        colbs.append(pl.multiple_of(colb, 128))

    copies = []
    for k in range(RPW):
        for i in range(RPW):
            for j in range(8):
                col = colbs[k] + (lane + j * 16)
                wbuf[k, i, pl.ds(j * 16, 16)] = jnp.where(
                    col == toks[i], jnp.float32(5.0), jnp.float32(0.0)
                )
        dst = out_hbm.at[pl.ds(base, RPW), pl.ds(colbs[k], 128)]
        copies.append(pltpu.async_copy(wbuf.at[k], dst, wsem))
    for c in copies:
        c.wait()


def kernel(hidden_states):
    hs = hidden_states.reshape(ROWS, H)
    logits = _tc_zeros()
    buf = jax.new_ref(logits)
    _sc_scatter(hs, buf)
    return buf[...].reshape(B, S, V)


# speedup vs baseline: 1.0633x; 1.0159x over previous
import functools

import jax
import jax.numpy as jnp
from jax import lax
from jax.experimental import pallas as pl
from jax.experimental.pallas import tpu as pltpu
from jax.experimental.pallas import tpu_sc as plsc

B, S, H = 32, 8, 1024
V = 100000
ROWS = B * S
NC, NS = 2, 16
NW = NC * NS
RPW = ROWS // NW
ROW_BLK = 8
MAGIC = 12582912.0

_mesh = plsc.VectorSubcoreMesh(core_axis_name="c", subcore_axis_name="s")


def _zero_body(out_ref):
    out_ref[...] = jnp.zeros((ROW_BLK, V), jnp.float32)


_tc_zeros = pl.pallas_call(
    _zero_body,
    out_shape=jax.ShapeDtypeStruct((ROWS, V), jnp.float32),
    grid=(ROWS // ROW_BLK,),
    out_specs=pl.BlockSpec((ROW_BLK, V), lambda i: (i, 0)),
)


@functools.partial(
    pl.kernel,
    mesh=_mesh,
    out_type=(),
    scratch_types=[
        pltpu.VMEM((RPW, 128), jnp.float32),
        pltpu.VMEM((RPW, RPW, 128), jnp.float32),
        pltpu.SemaphoreType.DMA,
    ],
)
def _sc_scatter(hs_hbm, out_hbm, hsv, wbuf, wsem):
    wid = lax.axis_index("s") * NC + lax.axis_index("c")
    base = wid * RPW

    pltpu.sync_copy(hs_hbm.at[pl.ds(base, RPW), pl.ds(0, 128)], hsv)
    lane = lax.iota(jnp.int32, 16)
    toks = []
    colbs = []
    for r in range(RPW):
        x = hsv[r, pl.ds(0, 16)][0]
        rr = (x + MAGIC) - MAGIC
        rr = jnp.minimum(jnp.maximum(rr, 0.0), float(V - 1))
        tok = rr.astype(jnp.int32)
        toks.append(tok)
        colb = tok - jnp.bitwise_and(tok, 127)
        colbs.append(pl.multiple_of(colb, 128))

    copies = []
    for k in range(RPW):
        for i in range(RPW):
            for j in range(8):
                col = colbs[k] + (lane + j * 16)
                wbuf[k, i, pl.ds(j * 16, 16)] = jnp.where(
                    col == toks[i], jnp.float32(5.0), jnp.float32(0.0)
                )
        dst = out_hbm.at[pl.ds(base, RPW), pl.ds(colbs[k], 128)]
        copies.append(pltpu.async_copy(wbuf.at[k], dst, wsem))
    for c in copies:
        c.wait()


def kernel(hidden_states):
    hs = hidden_states.reshape(ROWS, H)
    logits = _tc_zeros()
    buf = jax.new_ref(logits)
    _sc_scatter(hs, buf)
    return buf[...].reshape(B, S, V)
